# baseline (speedup 1.0000x reference)
import jax
import jax.numpy as jnp
from jax.experimental import pallas as pl
from jax.experimental.pallas import tpu as pltpu

_NIN = 6
_NOUT = 6


def _pipeline_kernel(x_hbm, t_hbm, o_hbm, xbuf, obuf, tbuf, in_sem, out_sem, t_sem):
    batch = x_hbm.shape[0]

    tcopy = pltpu.make_async_copy(t_hbm, tbuf, t_sem)
    tcopy.start()

    def in_copy(b):
        slot = jax.lax.rem(b, _NIN)
        return pltpu.make_async_copy(x_hbm.at[b], xbuf.at[slot], in_sem.at[slot])

    def out_copy(b):
        slot = jax.lax.rem(b, _NOUT)
        return pltpu.make_async_copy(obuf.at[slot], o_hbm.at[b], out_sem.at[slot])

    for i in range(_NIN):
        in_copy(i).start()

    tcopy.wait()

    def step(b, carry):
        si = jax.lax.rem(b, _NIN)
        so = jax.lax.rem(b, _NOUT)
        in_copy(b).wait()

        @pl.when(b >= _NOUT)
        def _():
            out_copy(b - _NOUT).wait()

        obuf[so] = xbuf[si] + tbuf[...]
        out_copy(b).start()

        @pl.when(b + _NIN < batch)
        def _():
            in_copy(b + _NIN).start()

        return carry

    jax.lax.fori_loop(0, batch, step, 0)

    for i in range(_NOUT):
        out_copy(batch - _NOUT + i).wait()


def kernel(inputs, pos_table):
    batch, positions, dim = inputs.shape
    return pl.pallas_call(
        _pipeline_kernel,
        in_specs=[
            pl.BlockSpec(memory_space=pltpu.HBM),
            pl.BlockSpec(memory_space=pltpu.HBM),
        ],
        out_specs=pl.BlockSpec(memory_space=pltpu.HBM),
        out_shape=jax.ShapeDtypeStruct(inputs.shape, inputs.dtype),
        scratch_shapes=[
            pltpu.VMEM((_NIN, positions, dim), inputs.dtype),
            pltpu.VMEM((_NOUT, positions, dim), inputs.dtype),
            pltpu.VMEM((positions, dim), pos_table.dtype),
            pltpu.SemaphoreType.DMA((_NIN,)),
            pltpu.SemaphoreType.DMA((_NOUT,)),
            pltpu.SemaphoreType.DMA,
        ],
    )(inputs, pos_table)
